# baseline (speedup 1.0000x reference)
import functools

import jax
import jax.numpy as jnp
from jax import lax
from jax.experimental import pallas as pl
from jax.experimental.pallas import tpu as pltpu
from jax.experimental.pallas import tpu_sc as plsc

_NC = 2
_NS = 16
_NW = _NC * _NS

_CHUNK = 128
_PAD = 128
_TCOLS = 8192


def _transpose_body(dim, in_ref, out_ref):
    x = in_ref[...]
    y = x.T
    out_ref[:, :dim] = y
    out_ref[:, dim:] = jnp.zeros_like(out_ref[:, dim:])


def _tc_transpose(table_t, vocab_pad):
    dim, vocab = table_t.shape
    grid = (vocab_pad // _TCOLS,)
    return pl.pallas_call(
        functools.partial(_transpose_body, dim),
        grid=grid,
        in_specs=[pl.BlockSpec((dim, _TCOLS), lambda i: (0, i))],
        out_specs=pl.BlockSpec((_TCOLS, _PAD), lambda i: (i, 0)),
        out_shape=jax.ShapeDtypeStruct((vocab_pad, _PAD), jnp.float32),
    )(table_t)


def _gather_body(n_tok, n_chunks, sen_hbm, table_hbm, out_hbm,
                 idx_v, rows0, rows1, sem0, sem1):
    wid = lax.axis_index("s") * _NC + lax.axis_index("c")
    base = wid * n_tok

    pltpu.sync_copy(sen_hbm.at[pl.ds(base, n_tok)], idx_v)

    pltpu.async_copy(table_hbm.at[idx_v.at[pl.ds(0, _CHUNK)]], rows0, sem0)
    pltpu.async_copy(table_hbm.at[idx_v.at[pl.ds(_CHUNK, _CHUNK)]], rows1, sem1)

    def body(i, carry):
        for b, (buf, sem) in enumerate(((rows0, sem0), (rows1, sem1))):
            g = 2 * i + b
            pltpu.make_async_copy(
                table_hbm.at[idx_v.at[pl.ds(g * _CHUNK, _CHUNK)]], buf, sem
            ).wait()
            pltpu.sync_copy(buf, out_hbm.at[pl.ds(base + g * _CHUNK, _CHUNK)])

            @pl.when(g + 2 < n_chunks)
            def _():
                pltpu.async_copy(
                    table_hbm.at[idx_v.at[pl.ds((g + 2) * _CHUNK, _CHUNK)]],
                    buf, sem)
        return carry

    lax.fori_loop(0, n_chunks // 2, body, 0)


@functools.partial(jax.jit, static_argnames=("n_tok_total",))
def _sc_gather(sen_flat, table_pad, n_tok_total):
    n_tok = n_tok_total // _NW
    n_chunks = n_tok // _CHUNK
    mesh = plsc.VectorSubcoreMesh(core_axis_name="c", subcore_axis_name="s")
    return pl.kernel(
        functools.partial(_gather_body, n_tok, n_chunks),
        out_type=jax.ShapeDtypeStruct((n_tok_total, _PAD), jnp.float32),
        mesh=mesh,
        scratch_types=[
            pltpu.VMEM((n_tok,), jnp.int32),
            pltpu.VMEM((_CHUNK, _PAD), jnp.float32),
            pltpu.VMEM((_CHUNK, _PAD), jnp.float32),
            pltpu.SemaphoreType.DMA,
            pltpu.SemaphoreType.DMA,
        ],
    )(sen_flat, table_pad)


_LBLK = 8


def _ln_body(dim, b, rows_ref, pos_ref, gamma_ref, beta_ref, *rest):
    out_ref = rest[-1]
    for j in range(_LBLK):
        x = rows_ref[pl.ds(j * b, b), :] + pos_ref[j:j + 1, :]
        s = jnp.sum(x, axis=-1, keepdims=True)
        sq = jnp.sum(x * x, axis=-1, keepdims=True)
        mean = s / dim
        var = sq / dim - mean * mean
        y = ((x - mean) * lax.rsqrt(var + 1e-6)) * gamma_ref[...] + beta_ref[...]
        yt = y.T
        out_ref[:, j, :] = yt[:dim, :]


def _tc_layernorm(gathered, pos_pad, gamma_pad, beta_pad, out_acc,
                  b, l, l_cnt, l_off):
    dim = 100
    grid = (l_cnt // _LBLK,)
    blk0 = l_off // _LBLK
    in_specs = [
        pl.BlockSpec((_LBLK * b, _PAD), lambda i: (i, 0)),
        pl.BlockSpec((_LBLK, _PAD), lambda i: (i + blk0, 0)),
        pl.BlockSpec((1, _PAD), lambda i: (0, 0)),
        pl.BlockSpec((1, _PAD), lambda i: (0, 0)),
    ]
    args = [gathered, pos_pad, gamma_pad, beta_pad]
    aliases = {}
    if out_acc is not None:
        in_specs.append(pl.BlockSpec((dim, _LBLK, b), lambda i: (0, i + blk0, 0)))
        args.append(out_acc)
        aliases = {4: 0}
    return pl.pallas_call(
        functools.partial(_ln_body, dim, b),
        grid=grid,
        in_specs=in_specs,
        out_specs=pl.BlockSpec((dim, _LBLK, b), lambda i: (0, i + blk0, 0)),
        out_shape=jax.ShapeDtypeStruct((dim, l, b), jnp.float32),
        input_output_aliases=aliases,
    )(*args)


def kernel(sen, table, pos_emb, gamma, beta):
    b, l = sen.shape
    vocab, dim = table.shape
    vocab_pad = ((vocab + _TCOLS - 1) // _TCOLS) * _TCOLS
    sen_flat = sen.T.reshape(-1).astype(jnp.int32)
    table_pad = _tc_transpose(table.T, vocab_pad)
    pad = ((0, 0), (0, _PAD - dim))
    pos_pad = jnp.pad(pos_emb[:l], pad)
    gamma_pad = jnp.pad(gamma.reshape(1, dim), pad)
    beta_pad = jnp.pad(beta.reshape(1, dim), pad)

    splits = (96, 104) if l == 200 else (l,)
    gs = []
    off = 0
    for cnt in splits:
        gs.append(_sc_gather(sen_flat[off * b:(off + cnt) * b], table_pad,
                             cnt * b))
        off += cnt
    out_acc = None
    off = 0
    for cnt, g in zip(splits, gs):
        out_acc = _tc_layernorm(g, pos_pad, gamma_pad, beta_pad, out_acc,
                                b, l, cnt, off)
        off += cnt
    return out_acc.transpose(2, 1, 0)


# speedup vs baseline: 1.0104x; 1.0104x over previous
import functools

import jax
import jax.numpy as jnp
from jax import lax
from jax.experimental import pallas as pl
from jax.experimental.pallas import tpu as pltpu
from jax.experimental.pallas import tpu_sc as plsc

_NC = 2
_NS = 16
_NW = _NC * _NS

_CHUNK = 128
_PAD = 128
_TCOLS = 16384


def _transpose_body(dim, in_ref, out_ref):
    x = in_ref[...]
    y = x.T
    out_ref[:, :dim] = y
    out_ref[:, dim:] = jnp.zeros_like(out_ref[:, dim:])


def _tc_transpose(table_t, vocab_pad):
    dim, vocab = table_t.shape
    grid = (vocab_pad // _TCOLS,)
    return pl.pallas_call(
        functools.partial(_transpose_body, dim),
        grid=grid,
        in_specs=[pl.BlockSpec((dim, _TCOLS), lambda i: (0, i))],
        out_specs=pl.BlockSpec((_TCOLS, _PAD), lambda i: (i, 0)),
        out_shape=jax.ShapeDtypeStruct((vocab_pad, _PAD), jnp.float32),
    )(table_t)


def _gather_body(n_tok, n_chunks, sen_hbm, table_hbm, out_hbm,
                 idx_v, rows0, rows1, sem0, sem1):
    wid = lax.axis_index("s") * _NC + lax.axis_index("c")
    base = wid * n_tok

    pltpu.sync_copy(sen_hbm.at[pl.ds(base, n_tok)], idx_v)

    pltpu.async_copy(table_hbm.at[idx_v.at[pl.ds(0, _CHUNK)]], rows0, sem0)
    pltpu.async_copy(table_hbm.at[idx_v.at[pl.ds(_CHUNK, _CHUNK)]], rows1, sem1)

    def body(i, carry):
        for b, (buf, sem) in enumerate(((rows0, sem0), (rows1, sem1))):
            g = 2 * i + b
            pltpu.make_async_copy(
                table_hbm.at[idx_v.at[pl.ds(g * _CHUNK, _CHUNK)]], buf, sem
            ).wait()
            pltpu.sync_copy(buf, out_hbm.at[pl.ds(base + g * _CHUNK, _CHUNK)])

            @pl.when(g + 2 < n_chunks)
            def _():
                pltpu.async_copy(
                    table_hbm.at[idx_v.at[pl.ds((g + 2) * _CHUNK, _CHUNK)]],
                    buf, sem)
        return carry

    lax.fori_loop(0, n_chunks // 2, body, 0)


@functools.partial(jax.jit, static_argnames=("n_tok_total",))
def _sc_gather(sen_flat, table_pad, n_tok_total):
    n_tok = n_tok_total // _NW
    n_chunks = n_tok // _CHUNK
    mesh = plsc.VectorSubcoreMesh(core_axis_name="c", subcore_axis_name="s")
    return pl.kernel(
        functools.partial(_gather_body, n_tok, n_chunks),
        out_type=jax.ShapeDtypeStruct((n_tok_total, _PAD), jnp.float32),
        mesh=mesh,
        scratch_types=[
            pltpu.VMEM((n_tok,), jnp.int32),
            pltpu.VMEM((_CHUNK, _PAD), jnp.float32),
            pltpu.VMEM((_CHUNK, _PAD), jnp.float32),
            pltpu.SemaphoreType.DMA,
            pltpu.SemaphoreType.DMA,
        ],
    )(sen_flat, table_pad)


_LBLK = 8


def _ln_body(dim, b, rows_ref, pos_ref, gamma_ref, beta_ref, *rest):
    out_ref = rest[-1]
    for j in range(_LBLK):
        x = rows_ref[pl.ds(j * b, b), :] + pos_ref[j:j + 1, :]
        s = jnp.sum(x, axis=-1, keepdims=True)
        sq = jnp.sum(x * x, axis=-1, keepdims=True)
        mean = s / dim
        var = sq / dim - mean * mean
        y = ((x - mean) * lax.rsqrt(var + 1e-6)) * gamma_ref[...] + beta_ref[...]
        yt = y.T
        out_ref[:, j, :] = yt[:dim, :]


def _tc_layernorm(gathered, pos_pad, gamma_pad, beta_pad, out_acc,
                  b, l, l_cnt, l_off):
    dim = 100
    grid = (l_cnt // _LBLK,)
    blk0 = l_off // _LBLK
    in_specs = [
        pl.BlockSpec((_LBLK * b, _PAD), lambda i: (i, 0)),
        pl.BlockSpec((_LBLK, _PAD), lambda i: (i + blk0, 0)),
        pl.BlockSpec((1, _PAD), lambda i: (0, 0)),
        pl.BlockSpec((1, _PAD), lambda i: (0, 0)),
    ]
    args = [gathered, pos_pad, gamma_pad, beta_pad]
    aliases = {}
    if out_acc is not None:
        in_specs.append(pl.BlockSpec((dim, _LBLK, b), lambda i: (0, i + blk0, 0)))
        args.append(out_acc)
        aliases = {4: 0}
    return pl.pallas_call(
        functools.partial(_ln_body, dim, b),
        grid=grid,
        in_specs=in_specs,
        out_specs=pl.BlockSpec((dim, _LBLK, b), lambda i: (0, i + blk0, 0)),
        out_shape=jax.ShapeDtypeStruct((dim, l, b), jnp.float32),
        input_output_aliases=aliases,
    )(*args)


def kernel(sen, table, pos_emb, gamma, beta):
    b, l = sen.shape
    vocab, dim = table.shape
    vocab_pad = ((vocab + _TCOLS - 1) // _TCOLS) * _TCOLS
    sen_flat = sen.T.reshape(-1).astype(jnp.int32)
    table_pad = _tc_transpose(table.T, vocab_pad)
    pad = ((0, 0), (0, _PAD - dim))
    pos_pad = jnp.pad(pos_emb[:l], pad)
    gamma_pad = jnp.pad(gamma.reshape(1, dim), pad)
    beta_pad = jnp.pad(beta.reshape(1, dim), pad)

    splits = (96, 104) if l == 200 else (l,)
    gs = []
    off = 0
    for cnt in splits:
        gs.append(_sc_gather(sen_flat[off * b:(off + cnt) * b], table_pad,
                             cnt * b))
        off += cnt
    out_acc = None
    off = 0
    for cnt, g in zip(splits, gs):
        out_acc = _tc_layernorm(g, pos_pad, gamma_pad, beta_pad, out_acc,
                                b, l, cnt, off)
        off += cnt
    return out_acc.transpose(2, 1, 0)
